# baseline (speedup 1.0000x reference)
import functools

import jax
import jax.numpy as jnp
import numpy as np
from jax.experimental import pallas as pl
from jax.experimental.pallas import tpu as pltpu

_B, _T, _D = 2, 2048, 1024
_K = 512
_NT = jax.lax.dot_general

_SIGN = int(np.int32(np.uint32(0x80000000)))


def _attn_kernel(h_ref, wq_ref, bq_ref, wk_ref, bk_ref, wv_ref, bv_ref, out_ref):
    h = h_ref[0]
    h_last = h[_T - 1:_T, :]
    f32 = jnp.float32
    q = _NT(h_last, wq_ref[...], (((1,), (1,)), ((), ())),
            preferred_element_type=f32) + bq_ref[...]
    k = _NT(h, wk_ref[...], (((1,), (1,)), ((), ())),
            preferred_element_type=f32) + bk_ref[...]
    s = _NT(q, k, (((1,), (1,)), ((), ())),
            preferred_element_type=f32) * f32(1.0 / np.sqrt(_D))

    xi = jax.lax.bitcast_convert_type(s, jnp.int32)
    key = xi ^ ((xi >> 31) & jnp.int32(0x7FFFFFFF))
    kk = jnp.int32(_K)
    cnt_pos = jnp.sum((key >= 0).astype(jnp.int32))
    take_pos = cnt_pos >= kk
    prefix = jnp.where(take_pos, jnp.int32(0), jnp.int32(_SIGN))
    kk = jnp.where(take_pos, kk, kk - cnt_pos)
    for bit in range(30, -1, -1):
        m = jnp.int32(np.int32(np.uint32((0xFFFFFFFF << (bit + 1)) & 0xFFFFFFFF)))
        bitc = jnp.int32(1 << bit)
        cnt = jnp.sum(((key & (m | bitc)) == (prefix | bitc)).astype(jnp.int32))
        take = cnt >= kk
        prefix = jnp.where(take, prefix | bitc, prefix)
        kk = jnp.where(take, kk, kk - cnt)
    sel = key >= prefix

    mx = jnp.max(s)
    w = jnp.where(sel, jnp.exp(s - mx), f32(0.0))
    denom = jnp.sum(w)
    acc = _NT(w, h, (((1,), (0,)), ((), ())), preferred_element_type=f32)
    out = _NT(acc, wv_ref[...], (((1,), (1,)), ((), ())),
              preferred_element_type=f32) / denom + bv_ref[...]
    out_ref[0] = out


@jax.jit
def kernel(h, Wq, bq, Wk, bk, Wv, bv):
    bq2 = bq.reshape(1, _D)
    bk2 = bk.reshape(1, _D)
    bv2 = bv.reshape(1, _D)
    grid = (_B,)
    out = pl.pallas_call(
        _attn_kernel,
        grid=grid,
        in_specs=[
            pl.BlockSpec((1, _T, _D), lambda b: (b, 0, 0)),
            pl.BlockSpec((_D, _D), lambda b: (0, 0)),
            pl.BlockSpec((1, _D), lambda b: (0, 0)),
            pl.BlockSpec((_D, _D), lambda b: (0, 0)),
            pl.BlockSpec((1, _D), lambda b: (0, 0)),
            pl.BlockSpec((_D, _D), lambda b: (0, 0)),
            pl.BlockSpec((1, _D), lambda b: (0, 0)),
        ],
        out_specs=pl.BlockSpec((1, 1, _D), lambda b: (b, 0, 0)),
        out_shape=jax.ShapeDtypeStruct((_B, 1, _D), jnp.float32),
        compiler_params=pltpu.CompilerParams(
            dimension_semantics=("parallel",),
        ),
    )(h, Wq, bq2, Wk, bk2, Wv, bv2)
    return out.reshape(_B, _D)


# speedup vs baseline: 1.2238x; 1.2238x over previous
import jax
import jax.numpy as jnp
import numpy as np
from jax.experimental import pallas as pl
from jax.experimental.pallas import tpu as pltpu

_B, _T, _D = 2, 2048, 1024
_K = 512
_NT = jax.lax.dot_general

_SIGN = int(np.int32(np.uint32(0x80000000)))


def _attn_kernel(h_ref, wq_ref, bq_ref, wk_ref, bk_ref, wv_ref, bv_ref, out_ref):
    h = h_ref[0]
    h_last = h[_T - 1:_T, :]
    f32 = jnp.float32
    i32 = jnp.int32
    q = _NT(h_last, wq_ref[...], (((1,), (1,)), ((), ())),
            preferred_element_type=f32) + bq_ref[...]
    k = _NT(h, wk_ref[...], (((1,), (1,)), ((), ())),
            preferred_element_type=f32) + bk_ref[...]
    s = _NT(q, k, (((1,), (1,)), ((), ())),
            preferred_element_type=f32) * f32(1.0 / np.sqrt(_D))

    xi = jax.lax.bitcast_convert_type(s, i32)
    key = xi ^ ((xi >> 31) & i32(0x7FFFFFFF))
    ukey = jnp.reshape(key, (16, 128)) ^ i32(_SIGN)
    kk = jnp.full((1, 1), _K, dtype=i32)
    uprefix = jnp.zeros((1, 1), dtype=i32)
    for shift in range(28, -1, -4):
        hi_mask = i32(np.int32(np.uint32((0xFFFFFFFF << (shift + 4)) & 0xFFFFFFFF)))
        match = (ukey & hi_mask) == uprefix
        dg = (ukey >> shift) & i32(15)
        c_ge = [jnp.sum((match & (dg >= i32(d))).astype(i32), keepdims=True)
                for d in range(1, 16)]
        ind = [(c >= kk).astype(i32) for c in c_ge]
        digit = ind[0]
        for x in ind[1:]:
            digit = digit + x
        nxt = jnp.zeros((1, 1), dtype=i32)
        for c in c_ge:
            nxt = jnp.maximum(nxt, jnp.where(c < kk, c, i32(0)))
        kk = kk - nxt
        uprefix = uprefix | (digit * i32(1 << shift))
    sprefix = uprefix ^ i32(_SIGN)
    sel = key >= sprefix

    mx = jnp.max(s, keepdims=True)
    w = jnp.where(sel, jnp.exp(s - mx), f32(0.0))
    denom = jnp.sum(w, keepdims=True)
    acc = _NT(w, h, (((1,), (0,)), ((), ())), preferred_element_type=f32)
    out = _NT(acc, wv_ref[...], (((1,), (1,)), ((), ())),
              preferred_element_type=f32) / denom + bv_ref[...]
    out_ref[0] = out


@jax.jit
def kernel(h, Wq, bq, Wk, bk, Wv, bv):
    bq2 = bq.reshape(1, _D)
    bk2 = bk.reshape(1, _D)
    bv2 = bv.reshape(1, _D)
    grid = (_B,)
    out = pl.pallas_call(
        _attn_kernel,
        grid=grid,
        in_specs=[
            pl.BlockSpec((1, _T, _D), lambda b: (b, 0, 0)),
            pl.BlockSpec((_D, _D), lambda b: (0, 0)),
            pl.BlockSpec((1, _D), lambda b: (0, 0)),
            pl.BlockSpec((_D, _D), lambda b: (0, 0)),
            pl.BlockSpec((1, _D), lambda b: (0, 0)),
            pl.BlockSpec((_D, _D), lambda b: (0, 0)),
            pl.BlockSpec((1, _D), lambda b: (0, 0)),
        ],
        out_specs=pl.BlockSpec((1, 1, _D), lambda b: (b, 0, 0)),
        out_shape=jax.ShapeDtypeStruct((_B, 1, _D), jnp.float32),
        compiler_params=pltpu.CompilerParams(
            dimension_semantics=("arbitrary",),
        ),
    )(h, Wq, bq2, Wk, bk2, Wv, bv2)
    return out.reshape(_B, _D)


# speedup vs baseline: 1.3041x; 1.0657x over previous
import jax
import jax.numpy as jnp
import numpy as np
from jax.experimental import pallas as pl
from jax.experimental.pallas import tpu as pltpu

_B, _T, _D = 2, 2048, 1024
_K = 512
_NT = jax.lax.dot_general

_SIGN = int(np.int32(np.uint32(0x80000000)))


def _select_topk(s, i32):
    xi = jax.lax.bitcast_convert_type(s, i32)
    key = xi ^ ((xi >> 31) & i32(0x7FFFFFFF))
    ukey = jnp.reshape(key, (16, 128)) ^ i32(_SIGN)
    kk = jnp.full((1, 1), _K, dtype=i32)
    uprefix = jnp.zeros((1, 1), dtype=i32)
    for shift in range(28, -1, -4):
        hi_mask = i32(np.int32(np.uint32((0xFFFFFFFF << (shift + 4)) & 0xFFFFFFFF)))
        match = (ukey & hi_mask) == uprefix
        dg = (ukey >> shift) & i32(15)
        c_ge = [jnp.sum((match & (dg >= i32(d))).astype(i32), keepdims=True)
                for d in range(1, 16)]
        ind = [(c >= kk).astype(i32) for c in c_ge]
        digit = ind[0]
        for x in ind[1:]:
            digit = digit + x
        nxt = jnp.zeros((1, 1), dtype=i32)
        for c in c_ge:
            nxt = jnp.maximum(nxt, jnp.where(c < kk, c, i32(0)))
        kk = kk - nxt
        uprefix = uprefix | (digit * i32(1 << shift))
    sprefix = uprefix ^ i32(_SIGN)
    return key >= sprefix


def _attn_kernel(h_ref, wq_ref, bq_ref, wk_ref, bk_ref, wv_ref, bv_ref, out_ref):
    f32 = jnp.float32
    i32 = jnp.int32
    h0 = h_ref[0]
    h1 = h_ref[1]
    h_last = jnp.concatenate([h0[_T - 1:_T, :], h1[_T - 1:_T, :]], axis=0)
    q = _NT(h_last, wq_ref[...], (((1,), (1,)), ((), ())),
            preferred_element_type=f32) + bq_ref[...]
    scl = f32(1.0 / np.sqrt(_D))
    ss = []
    for b, hb in enumerate((h0, h1)):
        k = _NT(hb, wk_ref[...], (((1,), (1,)), ((), ())),
                preferred_element_type=f32) + bk_ref[...]
        ss.append(_NT(q[b:b + 1, :], k, (((1,), (1,)), ((), ())),
                      preferred_element_type=f32) * scl)
    sels = [_select_topk(s, i32) for s in ss]
    accs = []
    for b, hb in enumerate((h0, h1)):
        s, sel = ss[b], sels[b]
        mx = jnp.max(s, keepdims=True)
        w = jnp.where(sel, jnp.exp(s - mx), f32(0.0))
        denom = jnp.sum(w, keepdims=True)
        accs.append(_NT(w, hb, (((1,), (0,)), ((), ())),
                        preferred_element_type=f32) / denom)
    acc2 = jnp.concatenate(accs, axis=0)
    out = _NT(acc2, wv_ref[...], (((1,), (1,)), ((), ())),
              preferred_element_type=f32) + bv_ref[...]
    out_ref[...] = out


@jax.jit
def kernel(h, Wq, bq, Wk, bk, Wv, bv):
    bq2 = bq.reshape(1, _D)
    bk2 = bk.reshape(1, _D)
    bv2 = bv.reshape(1, _D)
    out = pl.pallas_call(
        _attn_kernel,
        grid=(1,),
        in_specs=[
            pl.BlockSpec((_B, _T, _D), lambda i: (0, 0, 0)),
            pl.BlockSpec((_D, _D), lambda i: (0, 0)),
            pl.BlockSpec((1, _D), lambda i: (0, 0)),
            pl.BlockSpec((_D, _D), lambda i: (0, 0)),
            pl.BlockSpec((1, _D), lambda i: (0, 0)),
            pl.BlockSpec((_D, _D), lambda i: (0, 0)),
            pl.BlockSpec((1, _D), lambda i: (0, 0)),
        ],
        out_specs=pl.BlockSpec((_B, _D), lambda i: (0, 0)),
        out_shape=jax.ShapeDtypeStruct((_B, _D), jnp.float32),
        compiler_params=pltpu.CompilerParams(
            dimension_semantics=("arbitrary",),
        ),
    )(h, Wq, bq2, Wk, bk2, Wv, bv2)
    return out
